# Initial kernel scaffold; baseline (speedup 1.0000x reference)
import jax
import jax.numpy as jnp
from jax.experimental import pallas as pl


def kernel(x, sub_edge_index, edge_index, batch, emb0, emb1, emb2, mm1_W, mm1_b, brW, brb, mm2_W, mm2_b, hW1, hb1, hW2, hb2):
    raise NotImplementedError("write your pallas kernel here")


# speedup vs baseline: 3.2330x; 3.2330x over previous
import functools

import jax
import jax.numpy as jnp
from jax import lax
from jax.experimental import pallas as pl
from jax.experimental.pallas import tpu as pltpu
from jax.experimental.pallas import tpu_sc as plsc

N = 50000
E = 800000
H = 96
G = 128

NR = 51200
RPT = NR // 16
FCC = 6
CH = 128
NCH = 394
EPT = NCH * CH
BN = 512
NBLK = NR // BN


@functools.cache
def _sc_segsum_kernel(nch, rpt, nracc):
    def body(h6, src6, dst3, zrows, mout, acc, sidx, dring,
             rows0, rows1, zbuf, sem0, sem1, dsm0, dsm1):
        c = lax.axis_index("c")
        s = lax.axis_index("s")
        row0 = s * rpt
        pltpu.sync_copy(zrows, zbuf)

        def _wait_rows(buf, sem):
            pltpu.make_async_copy(h6.at[pl.ds(0, CH)], buf, sem).wait()

        def _wait_didx(k, sem):
            pltpu.make_async_copy(dst3.at[s, 0], dring.at[k], sem).wait()

        def _issue(j, buf, sem, k, dsem):
            pltpu.async_copy(h6.at[sidx.at[j]], buf, sem)
            pltpu.async_copy(dst3.at[s, j], dring.at[k], dsem)

        for j in range(3):
            fc = 3 * c + j
            pltpu.sync_copy(src6.at[fc, s], sidx)

            if rpt >= CH:
                @pl.loop(0, rpt // CH)
                def _zero(z):
                    pltpu.sync_copy(zbuf, acc.at[pl.ds(row0 + z * CH, CH)])
            else:
                pltpu.sync_copy(zbuf.at[pl.ds(0, rpt)], acc.at[pl.ds(row0, rpt)])

            plsc.subcore_barrier()

            _issue(0, rows0, sem0, 0, dsm0)
            _issue(1, rows1, sem1, 1, dsm1)

            @pl.loop(0, nch - 2, step=2)
            def _main(jj):
                _wait_rows(rows0, sem0)
                _wait_didx(0, dsm0)
                pltpu.sync_copy(rows0, acc.at[dring.at[0]], add=True)
                _issue(jj + 2, rows0, sem0, 0, dsm0)
                _wait_rows(rows1, sem1)
                _wait_didx(1, dsm1)
                pltpu.sync_copy(rows1, acc.at[dring.at[1]], add=True)
                _issue(jj + 3, rows1, sem1, 1, dsm1)

            _wait_rows(rows0, sem0)
            _wait_didx(0, dsm0)
            pltpu.sync_copy(rows0, acc.at[dring.at[0]], add=True)
            _wait_rows(rows1, sem1)
            _wait_didx(1, dsm1)
            pltpu.sync_copy(rows1, acc.at[dring.at[1]], add=True)

            plsc.subcore_barrier()
            pltpu.sync_copy(acc.at[pl.ds(row0, rpt)],
                            mout.at[pl.ds(row0, rpt), pl.ds(fc * 16, 16)])
            plsc.subcore_barrier()

    return pl.kernel(
        body,
        out_type=jax.ShapeDtypeStruct((nracc, H), jnp.float32),
        mesh=plsc.VectorSubcoreMesh(core_axis_name="c", subcore_axis_name="s"),
        scratch_types=[
            pltpu.VMEM_SHARED((nracc, 16), jnp.float32),
            pltpu.VMEM((nch, CH), jnp.int32),
            pltpu.VMEM((2, CH), jnp.int32),
            pltpu.VMEM((CH, 16), jnp.float32),
            pltpu.VMEM((CH, 16), jnp.float32),
            pltpu.VMEM((CH, 16), jnp.float32),
            pltpu.SemaphoreType.DMA,
            pltpu.SemaphoreType.DMA,
            pltpu.SemaphoreType.DMA,
            pltpu.SemaphoreType.DMA,
        ],
        compiler_params=pltpu.CompilerParams(use_tc_tiling_on_sc=False),
    )


def _sc_segsum(h6, s6, d3, zrows):
    return _sc_segsum_kernel(NCH, RPT, NR)(h6, s6, d3, zrows)


NCHP = 34
NRP = 256


def _sc_pool(h6, s6, d3, zrows):
    return _sc_segsum_kernel(NCHP, NRP // 16, NRP)(h6, s6, d3, zrows)


def _prep_edges(ei):
    return _prep(ei[0], ei[1], E, NCH, N, NR)


def _prep(src, dst, etot, nch, trash_lo, trash_hi):
    ept = nch * CH
    perm = jnp.argsort(dst, stable=True)
    s = src[perm]
    ds = dst[perm]
    b = jnp.arange(1, 16, dtype=jnp.int32) * (etot // 16)
    p = jnp.searchsorted(ds, ds[b - 1], side='right').astype(jnp.int32)
    p = jnp.concatenate([jnp.zeros((1,), jnp.int32), p])
    q = jnp.concatenate([p[1:], jnp.array([etot], jnp.int32)])
    idx = p[:, None] + jnp.arange(ept, dtype=jnp.int32)[None, :]
    valid = idx < q[:, None]
    idxc = jnp.minimum(idx, etot - 1)
    st = jnp.where(valid, s[idxc], (idx * 37) % N)
    dt = jnp.where(valid, ds[idxc], trash_lo + idx % (trash_hi - trash_lo))
    src6 = (st * 6)[None, :, :] + jnp.arange(6, dtype=jnp.int32)[:, None, None]
    return src6.reshape(FCC, 16, nch, CH), dt.reshape(16, nch, CH)


def _emb_body(x_ref, e0_ref, e1_ref, e2_ref, o_ref):
    xv = x_ref[...]
    outs = []
    for k, (eref, v) in enumerate(((e0_ref, 16), (e1_ref, 32), (e2_ref, 128))):
        col = lax.broadcasted_iota(jnp.int32, (BN, v), 1)
        oh = (xv[:, k][:, None] == col).astype(jnp.float32)
        outs.append(jnp.dot(oh, eref[...], preferred_element_type=jnp.float32,
                precision=lax.Precision.HIGHEST))
    o_ref[...] = jnp.concatenate(outs, axis=1)


def _emb_call(xpad, emb0, emb1, emb2):
    return pl.pallas_call(
        _emb_body,
        grid=(NBLK,),
        in_specs=[
            pl.BlockSpec((BN, 8), lambda i: (i, 0)),
            pl.BlockSpec((16, 32), lambda i: (0, 0)),
            pl.BlockSpec((32, 32), lambda i: (0, 0)),
            pl.BlockSpec((128, 32), lambda i: (0, 0)),
        ],
        out_specs=pl.BlockSpec((BN, H), lambda i: (i, 0)),
        out_shape=jax.ShapeDtypeStruct((NR, H), jnp.float32),
    )(xpad, emb0, emb1, emb2)


def _bdot(a, b):
    return jnp.dot(a.astype(jnp.bfloat16), b.astype(jnp.bfloat16),
                   preferred_element_type=jnp.float32)


def _xdot(a, b):
    return jnp.dot(a, b, preferred_element_type=jnp.float32,
                   precision=lax.Precision.HIGHEST)


def _layer_body(h_ref, m_ref, w_ref, b_ref, o_ref):
    t = _bdot(h_ref[...] + m_ref[...], w_ref[...])
    o_ref[...] = jnp.maximum(t + b_ref[...], 0.0)


def _layer_call(h, m, w, b):
    return pl.pallas_call(
        _layer_body,
        grid=(NBLK,),
        in_specs=[
            pl.BlockSpec((BN, H), lambda i: (i, 0)),
            pl.BlockSpec((BN, H), lambda i: (i, 0)),
            pl.BlockSpec((H, H), lambda i: (0, 0)),
            pl.BlockSpec((1, H), lambda i: (0, 0)),
        ],
        out_specs=pl.BlockSpec((BN, H), lambda i: (i, 0)),
        out_shape=jax.ShapeDtypeStruct((NR, H), jnp.float32),
    )(h, m, w, b)


def _layer_bridge_body(h_ref, m_ref, w_ref, b_ref, bw_ref, bb_ref, o_ref):
    t = jnp.maximum(_bdot(h_ref[...] + m_ref[...], w_ref[...]) + b_ref[...],
                    0.0)
    o_ref[...] = _bdot(t, bw_ref[...]) + bb_ref[...]


def _layer_bridge_call(h, m, w, b, bw, bb):
    return pl.pallas_call(
        _layer_bridge_body,
        grid=(NBLK,),
        in_specs=[
            pl.BlockSpec((BN, H), lambda i: (i, 0)),
            pl.BlockSpec((BN, H), lambda i: (i, 0)),
            pl.BlockSpec((H, H), lambda i: (0, 0)),
            pl.BlockSpec((1, H), lambda i: (0, 0)),
            pl.BlockSpec((H, H), lambda i: (0, 0)),
            pl.BlockSpec((1, H), lambda i: (0, 0)),
        ],
        out_specs=pl.BlockSpec((BN, H), lambda i: (i, 0)),
        out_shape=jax.ShapeDtypeStruct((NR, H), jnp.float32),
    )(h, m, w, b, bw, bb)


def _heads_body(xa_ref, w1_ref, b1_ref, w2_ref, b2_ref,
                o0_ref, o1_ref, o2_ref):
    xa = xa_ref[...]
    for k, o_ref in enumerate((o0_ref, o1_ref, o2_ref)):
        t = jnp.maximum(_bdot(xa, w1_ref[k]) + b1_ref[k][None, :], 0.0)
        o_ref[...] = _bdot(t, w2_ref[k]) + b2_ref[k][None, :]


def _heads_call(xa, hW1, hb1, hW2, hb2):
    oshape = jax.ShapeDtypeStruct((G, 1), jnp.float32)
    return pl.pallas_call(
        _heads_body,
        out_shape=(oshape, oshape, oshape),
    )(xa, hW1, hb1, hW2, hb2)


def kernel(x, sub_edge_index, edge_index, batch, emb0, emb1, emb2,
           mm1_W, mm1_b, brW, brb, mm2_W, mm2_b, hW1, hb1, hW2, hb2):
    xpad = jnp.zeros((NR, 8), jnp.int32).at[:N, :3].set(x)
    zrows = jnp.zeros((CH, 16), jnp.float32)
    s6a, d3a = _prep_edges(sub_edge_index)
    s6b, d3b = _prep_edges(edge_index)
    s6p, d3p = _prep(jnp.arange(N, dtype=jnp.int32), batch, N, NCHP, G, NRP)

    h = _emb_call(xpad, emb0, emb1, emb2)
    for i in range(3):
        m = _sc_segsum(h.reshape(NR * FCC, 16), s6a, d3a, zrows)
        if i < 2:
            h = _layer_call(h, m, mm1_W[i], mm1_b[i][None])
        else:
            h = _layer_bridge_call(h, m, mm1_W[2], mm1_b[2][None],
                                   brW, brb[None])
    for i in range(3):
        m = _sc_segsum(h.reshape(NR * FCC, 16), s6b, d3b, zrows)
        h = _layer_call(h, m, mm2_W[i], mm2_b[i][None])
    xa = _sc_pool(h.reshape(NR * FCC, 16), s6p, d3p, zrows)[:G]
    return _heads_call(xa, hW1, hb1, hW2, hb2)
